# Initial kernel scaffold; baseline (speedup 1.0000x reference)
import jax
import jax.numpy as jnp
from jax.experimental import pallas as pl


def kernel(pillar_features_bin_0, voxel_coords_bin_0, pillar_features_bin_1, voxel_coords_bin_1):
    raise NotImplementedError("write your pallas kernel here")


# speedup vs baseline: 1.0654x; 1.0654x over previous
import functools

import jax
import jax.numpy as jnp
from jax import lax
from jax.experimental import pallas as pl
from jax.experimental.pallas import tpu as pltpu
from jax.experimental.pallas import tpu_sc as plsc

NX = 512
NY = 512
C = 64
P = 100000
B = 2
SPATIAL = B * NY * NX

NC = 2
NS = 16
NW = NC * NS
PER_W = 3128
CH = 128
NCHUNK = 25


def _sc_scatter_body(f0, v0, f1, v1, canvas0, canvas1, coords_v, idx_v,
                     feat_v, sem):
  wid = lax.axis_index("s") * NC + lax.axis_index("c")
  base = wid * PER_W

  lanes = lax.iota(jnp.int32, 16)

  for feats, coords, canvas in ((f0, v0, canvas0), (f1, v1, canvas1)):

    def chunk_body(j, _, feats=feats, coords=coords, canvas=canvas):
      start = jnp.minimum(base + j * CH, P - CH)
      pltpu.sync_copy(coords.at[pl.ds(start * 4, CH * 4)], coords_v)
      pltpu.sync_copy(feats.at[pl.ds(start, CH)], feat_v)
      for g in range(CH // 16):
        rows4 = (g * 16 + lanes) * 4
        bcol = plsc.load_gather(coords_v, [rows4])
        ycol = plsc.load_gather(coords_v, [rows4 + 2])
        xcol = plsc.load_gather(coords_v, [rows4 + 3])
        flat = bcol * (NY * NX) + ycol * NX + xcol
        idx_v[pl.ds(g * 16, 16)] = flat
      pltpu.async_copy(feat_v, canvas.at[idx_v], sem).wait()
      return ()

    lax.fori_loop(0, NCHUNK, chunk_body, ())


def _sc_scatter(f0, v0, f1, v1, canvas0, canvas1):
  mesh = plsc.VectorSubcoreMesh(core_axis_name="c", subcore_axis_name="s")
  run = pl.kernel(
      _sc_scatter_body,
      out_type=(),
      mesh=mesh,
      compiler_params=pltpu.CompilerParams(
          needs_layout_passes=False, use_tc_tiling_on_sc=False),
      scratch_types=[
          pltpu.VMEM((CH * 4,), jnp.int32),
          pltpu.VMEM((CH,), jnp.int32),
          pltpu.VMEM((CH, C), jnp.float32),
          pltpu.SemaphoreType.DMA,
      ],
  )
  run(f0, v0, f1, v1, canvas0, canvas1)


S_BLK = 4096
NSB = NY * NX // S_BLK


def _tc_transpose_body(c0_ref, c1_ref, o0_ref, o1_ref):
  o0_ref[...] = jnp.transpose(c0_ref[...], (1, 0))[None]
  o1_ref[...] = jnp.transpose(c1_ref[...], (1, 0))[None]


def _tc_transpose(c0, c1):
  grid = (B, NSB)
  in_spec = pl.BlockSpec((S_BLK, C), lambda b, s: (b * NSB + s, 0))
  out_spec = pl.BlockSpec((1, C, S_BLK), lambda b, s: (b, 0, s))
  return pl.pallas_call(
      _tc_transpose_body,
      grid=grid,
      in_specs=[in_spec, in_spec],
      out_specs=[out_spec, out_spec],
      out_shape=[
          jax.ShapeDtypeStruct((B, C, NY * NX), jnp.float32),
          jax.ShapeDtypeStruct((B, C, NY * NX), jnp.float32),
      ],
      compiler_params=pltpu.CompilerParams(
          dimension_semantics=("parallel", "parallel"),
      ),
  )(c0, c1)


def kernel(pillar_features_bin_0, voxel_coords_bin_0, pillar_features_bin_1,
           voxel_coords_bin_1):
  canvas0 = jax.new_ref(jnp.zeros((SPATIAL, C), jnp.float32))
  canvas1 = jax.new_ref(jnp.zeros((SPATIAL, C), jnp.float32))
  _sc_scatter(pillar_features_bin_0, voxel_coords_bin_0.reshape(-1),
              pillar_features_bin_1, voxel_coords_bin_1.reshape(-1),
              canvas0, canvas1)
  t0, t1 = _tc_transpose(canvas0[...], canvas1[...])
  return (t0.reshape(B, C, NY, NX), t1.reshape(B, C, NY, NX))


# speedup vs baseline: 1.3294x; 1.2478x over previous
import functools

import jax
import jax.numpy as jnp
from jax import lax
from jax.experimental import pallas as pl
from jax.experimental.pallas import tpu as pltpu
from jax.experimental.pallas import tpu_sc as plsc

NX = 512
NY = 512
C = 64
P = 100000
B = 2
SPATIAL = B * NY * NX

NC = 2
NS = 16
NW = NC * NS
PER_W = 3128
CH = 128
NCHUNK = 25


def _sc_scatter_body(f0, v0, f1, v1, canvas0, canvas1, coords_v, idx_v,
                     feat_v, sem):
  wid = lax.axis_index("s") * NC + lax.axis_index("c")
  base = wid * PER_W

  lanes = lax.iota(jnp.int32, 16)

  for feats, coords, canvas in ((f0, v0, canvas0), (f1, v1, canvas1)):

    def chunk_body(j, _, feats=feats, coords=coords, canvas=canvas):
      start = jnp.minimum(base + j * CH, P - CH)
      pltpu.sync_copy(coords.at[pl.ds(start * 4, CH * 4)], coords_v)
      pltpu.sync_copy(feats.at[pl.ds(start, CH)], feat_v)
      for g in range(CH // 16):
        rows4 = (g * 16 + lanes) * 4
        bcol = plsc.load_gather(coords_v, [rows4])
        ycol = plsc.load_gather(coords_v, [rows4 + 2])
        xcol = plsc.load_gather(coords_v, [rows4 + 3])
        flat = (ycol * NX + xcol) * 2 + bcol
        idx_v[pl.ds(g * 16, 16)] = flat
      pltpu.async_copy(feat_v, canvas.at[idx_v], sem).wait()
      return ()

    lax.fori_loop(0, NCHUNK, chunk_body, ())


def _sc_scatter(f0, v0, f1, v1, canvas0, canvas1):
  mesh = plsc.VectorSubcoreMesh(core_axis_name="c", subcore_axis_name="s")
  run = pl.kernel(
      _sc_scatter_body,
      out_type=(),
      mesh=mesh,
      compiler_params=pltpu.CompilerParams(
          needs_layout_passes=False, use_tc_tiling_on_sc=False),
      scratch_types=[
          pltpu.VMEM((CH * 4,), jnp.int32),
          pltpu.VMEM((CH,), jnp.int32),
          pltpu.VMEM((CH, C), jnp.float32),
          pltpu.SemaphoreType.DMA,
      ],
  )
  run(f0, v0, f1, v1, canvas0, canvas1)


S_BLK = 4096
NSB = NY * NX // S_BLK


def _tc_transpose_body(c0_ref, c1_ref, o0_ref, o1_ref):
  for c_ref, o_ref in ((c0_ref, o0_ref), (c1_ref, o1_ref)):
    x = c_ref[...]
    o_ref[0] = jnp.transpose(x[:, :C], (1, 0))
    o_ref[1] = jnp.transpose(x[:, C:], (1, 0))


def _tc_transpose(c0, c1):
  grid = (NSB,)
  in_spec = pl.BlockSpec((S_BLK, 2 * C), lambda s: (s, 0))
  out_spec = pl.BlockSpec((B, C, S_BLK), lambda s: (0, 0, s))
  return pl.pallas_call(
      _tc_transpose_body,
      grid=grid,
      in_specs=[in_spec, in_spec],
      out_specs=[out_spec, out_spec],
      out_shape=[
          jax.ShapeDtypeStruct((B, C, NY * NX), jnp.float32),
          jax.ShapeDtypeStruct((B, C, NY * NX), jnp.float32),
      ],
      compiler_params=pltpu.CompilerParams(
          dimension_semantics=("parallel",),
      ),
  )(c0, c1)


def kernel(pillar_features_bin_0, voxel_coords_bin_0, pillar_features_bin_1,
           voxel_coords_bin_1):
  canvas0 = jax.new_ref(jnp.zeros((SPATIAL, C), jnp.float32))
  canvas1 = jax.new_ref(jnp.zeros((SPATIAL, C), jnp.float32))
  _sc_scatter(pillar_features_bin_0, voxel_coords_bin_0.reshape(-1),
              pillar_features_bin_1, voxel_coords_bin_1.reshape(-1),
              canvas0, canvas1)
  t0, t1 = _tc_transpose(canvas0[...].reshape(NY * NX, 2 * C),
                         canvas1[...].reshape(NY * NX, 2 * C))
  return (t0.reshape(B, C, NY, NX), t1.reshape(B, C, NY, NX))


# speedup vs baseline: 1.4598x; 1.0981x over previous
import functools

import jax
import jax.numpy as jnp
from jax import lax
from jax.experimental import pallas as pl
from jax.experimental.pallas import tpu as pltpu
from jax.experimental.pallas import tpu_sc as plsc

NX = 512
NY = 512
C = 64
P = 100000
B = 2
SPATIAL = B * NY * NX

NC = 2
NS = 16
NW = NC * NS
PER_W = 3128
CH = 128
NCHUNK = 25


def _sc_scatter_body(f0, v0, f1, v1, canvas0, canvas1, coords_v, idx_v,
                     feat_v, sem):
  wid = lax.axis_index("s") * NC + lax.axis_index("c")
  base = wid * PER_W

  lanes = lax.iota(jnp.int32, 16)

  for feats, coords, canvas in ((f0, v0, canvas0), (f1, v1, canvas1)):

    def chunk_body(j, _, feats=feats, coords=coords, canvas=canvas):
      start = jnp.minimum(base + j * CH, P - CH)
      pltpu.sync_copy(coords.at[pl.ds(start * 4, CH * 4)], coords_v)
      pltpu.sync_copy(feats.at[pl.ds(start, CH)], feat_v)
      for g in range(CH // 16):
        rows4 = (g * 16 + lanes) * 4
        bcol = plsc.load_gather(coords_v, [rows4])
        ycol = plsc.load_gather(coords_v, [rows4 + 2])
        xcol = plsc.load_gather(coords_v, [rows4 + 3])
        flat = (ycol * NX + xcol) * 2 + bcol
        idx_v[pl.ds(g * 16, 16)] = flat
      pltpu.async_copy(feat_v, canvas.at[idx_v], sem).wait()
      return ()

    lax.fori_loop(0, NCHUNK, chunk_body, ())


def _sc_scatter(f0, v0, f1, v1, canvas0, canvas1):
  mesh = plsc.VectorSubcoreMesh(core_axis_name="c", subcore_axis_name="s")
  run = pl.kernel(
      _sc_scatter_body,
      out_type=(),
      mesh=mesh,
      compiler_params=pltpu.CompilerParams(
          needs_layout_passes=False, use_tc_tiling_on_sc=False),
      scratch_types=[
          pltpu.VMEM((CH * 4,), jnp.int32),
          pltpu.VMEM((CH,), jnp.int32),
          pltpu.VMEM((CH, C), jnp.float32),
          pltpu.SemaphoreType.DMA,
      ],
  )
  run(f0, v0, f1, v1, canvas0, canvas1)


S_BLK = 4096
NSB = NY * NX // S_BLK


Y_BLK = S_BLK // NX


def _tc_transpose_body(c0_ref, c1_ref, o0_ref, o1_ref):
  for c_ref, o_ref in ((c0_ref, o0_ref), (c1_ref, o1_ref)):
    x = c_ref[...]
    for b in range(B):
      half = x[:, b * C:(b + 1) * C]
      for yy in range(Y_BLK):
        o_ref[b, :, yy, :] = jnp.transpose(
            half[yy * NX:(yy + 1) * NX, :], (1, 0))


def _tc_transpose(c0, c1):
  grid = (NSB,)
  in_spec = pl.BlockSpec((S_BLK, 2 * C), lambda s: (s, 0))
  out_spec = pl.BlockSpec((B, C, Y_BLK, NX), lambda s: (0, 0, s, 0))
  return pl.pallas_call(
      _tc_transpose_body,
      grid=grid,
      in_specs=[in_spec, in_spec],
      out_specs=[out_spec, out_spec],
      out_shape=[
          jax.ShapeDtypeStruct((B, C, NY, NX), jnp.float32),
          jax.ShapeDtypeStruct((B, C, NY, NX), jnp.float32),
      ],
      compiler_params=pltpu.CompilerParams(
          dimension_semantics=("parallel",),
      ),
  )(c0, c1)


def kernel(pillar_features_bin_0, voxel_coords_bin_0, pillar_features_bin_1,
           voxel_coords_bin_1):
  zeros = jnp.zeros((NY * NX, 2 * C), jnp.float32)
  canvas0 = jax.new_ref(zeros.reshape(SPATIAL, C))
  canvas1 = jax.new_ref(zeros.reshape(SPATIAL, C))
  _sc_scatter(pillar_features_bin_0, voxel_coords_bin_0.reshape(-1),
              pillar_features_bin_1, voxel_coords_bin_1.reshape(-1),
              canvas0, canvas1)
  return _tc_transpose(canvas0[...].reshape(NY * NX, 2 * C),
                       canvas1[...].reshape(NY * NX, 2 * C))


# speedup vs baseline: 3.5889x; 2.4585x over previous
import jax
import jax.numpy as jnp
from jax import lax
from jax.experimental import pallas as pl
from jax.experimental.pallas import tpu as pltpu
from jax.experimental.pallas import tpu_sc as plsc
from jax._src.pallas import mpmd as _mpmd

NX = 512
NY = 512
C = 64
P = 100000
B = 2
SPATIAL = B * NY * NX
NBIN = 2
CANVAS_ROWS = NBIN * SPATIAL

NC = 2
NS = 16
NW = NC * NS
PER_W = 3128
CH = 128
NCHUNK = 25


def _sc_scatter_body(f0, v0, f1, v1, zeros_in, canvas, coords_v, idx_v,
                     feat_v, sem):
  del zeros_in
  wid = lax.axis_index("s") * NC + lax.axis_index("c")
  base = wid * PER_W

  lanes = lax.iota(jnp.int32, 16)

  for bin_idx, (feats, coords) in enumerate(((f0, v0), (f1, v1))):
    bin_off = bin_idx * SPATIAL

    def chunk_body(j, _, feats=feats, coords=coords, bin_off=bin_off):
      start = jnp.minimum(base + j * CH, P - CH)
      pltpu.sync_copy(coords.at[pl.ds(start * 4, CH * 4)], coords_v)
      pltpu.sync_copy(feats.at[pl.ds(start, CH)], feat_v)
      for g in range(CH // 16):
        rows4 = (g * 16 + lanes) * 4
        bcol = plsc.load_gather(coords_v, [rows4])
        ycol = plsc.load_gather(coords_v, [rows4 + 2])
        xcol = plsc.load_gather(coords_v, [rows4 + 3])
        flat = (ycol * NX + xcol) * 2 + bcol + bin_off
        idx_v[pl.ds(g * 16, 16)] = flat
      pltpu.async_copy(feat_v, canvas.at[idx_v], sem).wait()
      return ()

    lax.fori_loop(0, NCHUNK, chunk_body, ())


def _sc_scatter(f0, v0, f1, v1, canvas_zeros):
  mesh = plsc.VectorSubcoreMesh(core_axis_name="c", subcore_axis_name="s")
  run = _mpmd._mpmd_map(
      [(mesh, _sc_scatter_body)],
      jax.ShapeDtypeStruct((CANVAS_ROWS, C), jnp.float32),
      input_output_aliases={4: 0},
      compiler_params=pltpu.CompilerParams(
          needs_layout_passes=False, use_tc_tiling_on_sc=False),
      scratch_types=[
          pltpu.VMEM((CH * 4,), jnp.int32),
          pltpu.VMEM((CH,), jnp.int32),
          pltpu.VMEM((CH, C), jnp.float32),
          pltpu.SemaphoreType.DMA,
      ],
  )
  return run(f0, v0, f1, v1, canvas_zeros)


S_BLK = 4096
NSB = NY * NX // S_BLK
Y_BLK = S_BLK // NX


def _tc_transpose_body(c0_ref, c1_ref, o0_ref, o1_ref):
  for c_ref, o_ref in ((c0_ref, o0_ref), (c1_ref, o1_ref)):
    x = c_ref[...]
    for b in range(B):
      half = x[:, b * C:(b + 1) * C]
      for yy in range(Y_BLK):
        o_ref[b, :, yy, :] = jnp.transpose(
            half[yy * NX:(yy + 1) * NX, :], (1, 0))


def _tc_transpose(canvas2d):
  grid = (NSB,)
  in_spec0 = pl.BlockSpec((S_BLK, 2 * C), lambda s: (s, 0))
  in_spec1 = pl.BlockSpec((S_BLK, 2 * C), lambda s: (s + NSB, 0))
  out_spec = pl.BlockSpec((B, C, Y_BLK, NX), lambda s: (0, 0, s, 0))
  return pl.pallas_call(
      _tc_transpose_body,
      grid=grid,
      in_specs=[in_spec0, in_spec1],
      out_specs=[out_spec, out_spec],
      out_shape=[
          jax.ShapeDtypeStruct((B, C, NY, NX), jnp.float32),
          jax.ShapeDtypeStruct((B, C, NY, NX), jnp.float32),
      ],
      compiler_params=pltpu.CompilerParams(
          dimension_semantics=("parallel",),
      ),
  )(canvas2d, canvas2d)


def kernel(pillar_features_bin_0, voxel_coords_bin_0, pillar_features_bin_1,
           voxel_coords_bin_1):
  zeros = jnp.zeros((NBIN * NY * NX, 2 * C), jnp.float32)
  canvas = _sc_scatter(pillar_features_bin_0, voxel_coords_bin_0.reshape(-1),
                       pillar_features_bin_1, voxel_coords_bin_1.reshape(-1),
                       zeros.reshape(CANVAS_ROWS, C))
  return _tc_transpose(canvas.reshape(NBIN * NY * NX, 2 * C))


# speedup vs baseline: 3.7018x; 1.0315x over previous
import jax
import jax.numpy as jnp
from jax import lax
from jax.experimental import pallas as pl
from jax.experimental.pallas import tpu as pltpu
from jax.experimental.pallas import tpu_sc as plsc
from jax._src.pallas import mpmd as _mpmd

NX = 512
NY = 512
C = 64
P = 100000
B = 2
SPATIAL = B * NY * NX
NBIN = 2
CANVAS_ROWS = NBIN * SPATIAL

NC = 2
NS = 16
NW = NC * NS
PER_W = 3128
CH = 128
NCHUNK = 25


PREP_BLK = 4096
PREP_GRID = -(-P // PREP_BLK)


def _prep_body(v0_ref, v1_ref, i0_ref, i1_ref):
  lane = lax.broadcasted_iota(jnp.int32, (1, 4), 1)
  w = ((lane == 0).astype(jnp.int32) + (lane == 2).astype(jnp.int32) * (2 * NX)
       + (lane == 3).astype(jnp.int32) * 2)
  for v_ref, i_ref, off in ((v0_ref, i0_ref, 0), (v1_ref, i1_ref, SPATIAL)):
    i_ref[...] = jnp.sum(v_ref[...] * w, axis=1) + off


def _tc_prep(v0, v1):
  v_spec = pl.BlockSpec((PREP_BLK, 4), lambda s: (s, 0))
  i_spec = pl.BlockSpec((PREP_BLK,), lambda s: (s,))
  return pl.pallas_call(
      _prep_body,
      grid=(PREP_GRID,),
      in_specs=[v_spec, v_spec],
      out_specs=[i_spec, i_spec],
      out_shape=[
          jax.ShapeDtypeStruct((P,), jnp.int32),
          jax.ShapeDtypeStruct((P,), jnp.int32),
      ],
      compiler_params=pltpu.CompilerParams(
          dimension_semantics=("parallel",),
      ),
  )(v0, v1)


def _sc_scatter_body(f0, i0, f1, i1, zeros_in, canvas,
                     idx_v, feat_v, sem_l, sem_s):
  del zeros_in
  wid = lax.axis_index("s") * NC + lax.axis_index("c")
  base = wid * PER_W

  def start_of(j):
    return jnp.minimum(base + j * CH, P - CH)

  for feats, flats in ((f0, i0), (f1, i1)):
    def load(j, k, feats=feats, flats=flats):
      s = start_of(j)
      pltpu.async_copy(flats.at[pl.ds(s, CH)], idx_v.at[k], sem_l)
      pltpu.async_copy(feats.at[pl.ds(s, CH)], feat_v.at[k], sem_l)

    def wait_load(j, k, feats=feats, flats=flats):
      s = start_of(j)
      pltpu.make_async_copy(flats.at[pl.ds(s, CH)], idx_v.at[k], sem_l).wait()
      pltpu.make_async_copy(feats.at[pl.ds(s, CH)], feat_v.at[k], sem_l).wait()

    def scatter_and_wait(k):
      pltpu.async_copy(feat_v.at[k], canvas.at[idx_v.at[k]], sem_s).wait()

    load(0, 0)
    load(1, 1)
    for j in range(NCHUNK):
      k = j % 2
      wait_load(j, k)
      scatter_and_wait(k)
      if j + 2 < NCHUNK:
        load(j + 2, k)


def _sc_scatter(f0, i0, f1, i1, canvas_zeros):
  mesh = plsc.VectorSubcoreMesh(core_axis_name="c", subcore_axis_name="s")
  run = _mpmd._mpmd_map(
      [(mesh, _sc_scatter_body)],
      jax.ShapeDtypeStruct((CANVAS_ROWS, C), jnp.float32),
      input_output_aliases={4: 0},
      compiler_params=pltpu.CompilerParams(
          needs_layout_passes=False, use_tc_tiling_on_sc=False),
      scratch_types=[
          pltpu.VMEM((2, CH), jnp.int32),
          pltpu.VMEM((2, CH, C), jnp.float32),
          pltpu.SemaphoreType.DMA,
          pltpu.SemaphoreType.DMA,
      ],
  )
  return run(f0, i0, f1, i1, canvas_zeros)


S_BLK = 4096
NSB = NY * NX // S_BLK
Y_BLK = S_BLK // NX


def _tc_transpose_body(c0_ref, c1_ref, o0_ref, o1_ref):
  for c_ref, o_ref in ((c0_ref, o0_ref), (c1_ref, o1_ref)):
    x = c_ref[...]
    for b in range(B):
      half = x[:, b * C:(b + 1) * C]
      for yy in range(Y_BLK):
        o_ref[b, :, yy, :] = jnp.transpose(
            half[yy * NX:(yy + 1) * NX, :], (1, 0))


def _tc_transpose(canvas2d):
  in_spec0 = pl.BlockSpec((S_BLK, 2 * C), lambda s: (s, 0))
  in_spec1 = pl.BlockSpec((S_BLK, 2 * C), lambda s: (s + NSB, 0))
  out_spec = pl.BlockSpec((B, C, Y_BLK, NX), lambda s: (0, 0, s, 0))
  return pl.pallas_call(
      _tc_transpose_body,
      grid=(NSB,),
      in_specs=[in_spec0, in_spec1],
      out_specs=[out_spec, out_spec],
      out_shape=[
          jax.ShapeDtypeStruct((B, C, NY, NX), jnp.float32),
          jax.ShapeDtypeStruct((B, C, NY, NX), jnp.float32),
      ],
      compiler_params=pltpu.CompilerParams(
          dimension_semantics=("parallel",),
      ),
  )(canvas2d, canvas2d)


def kernel(pillar_features_bin_0, voxel_coords_bin_0, pillar_features_bin_1,
           voxel_coords_bin_1):
  i0, i1 = _tc_prep(voxel_coords_bin_0, voxel_coords_bin_1)
  zeros = jnp.zeros((NBIN * NY * NX, 2 * C), jnp.float32)
  canvas = _sc_scatter(pillar_features_bin_0, i0, pillar_features_bin_1, i1,
                       zeros.reshape(CANVAS_ROWS, C))
  return _tc_transpose(canvas.reshape(NBIN * NY * NX, 2 * C))


# speedup vs baseline: 3.8965x; 1.0526x over previous
import jax
import jax.numpy as jnp
from jax import lax
from jax.experimental import pallas as pl
from jax.experimental.pallas import tpu as pltpu
from jax.experimental.pallas import tpu_sc as plsc
from jax._src.pallas import mpmd as _mpmd

NX = 512
NY = 512
C = 64
P = 100000
B = 2
SPATIAL = B * NY * NX
NBIN = 2
CANVAS_ROWS = NBIN * SPATIAL

NC = 2
NS = 16
NW = NC * NS
PER_W = 3128
CH = 128
NCHUNK = 25


def _sc_scatter_body(f0, v0, f1, v1, zeros_in, canvas,
                     coords_v, idx_v, feat_v, sem_l, sem_s):
  del zeros_in
  wid = lax.axis_index("s") * NC + lax.axis_index("c")
  base = wid * PER_W
  lanes = lax.iota(jnp.int32, 16)

  def start_of(j):
    return jnp.minimum(base + j * CH, P - CH)

  for bin_idx, (feats, coords) in enumerate(((f0, v0), (f1, v1))):
    bin_off = bin_idx * SPATIAL

    def load(j, k, feats=feats, coords=coords):
      s = start_of(j)
      pltpu.async_copy(coords.at[pl.ds(s * 4, CH * 4)], coords_v.at[k], sem_l)
      pltpu.async_copy(feats.at[pl.ds(s, CH)], feat_v.at[k], sem_l)

    def wait_load(j, k, feats=feats, coords=coords):
      s = start_of(j)
      pltpu.make_async_copy(
          coords.at[pl.ds(s * 4, CH * 4)], coords_v.at[k], sem_l).wait()
      pltpu.make_async_copy(
          feats.at[pl.ds(s, CH)], feat_v.at[k], sem_l).wait()

    def compute_idx(k, bin_off=bin_off):
      cv = coords_v.at[k]
      for g in range(CH // 16):
        rows4 = (g * 16 + lanes) * 4
        bcol = plsc.load_gather(cv, [rows4])
        ycol = plsc.load_gather(cv, [rows4 + 2])
        xcol = plsc.load_gather(cv, [rows4 + 3])
        idx_v[k, pl.ds(g * 16, 16)] = (ycol * NX + xcol) * 2 + bcol + bin_off

    def scatter_and_wait(k):
      pltpu.async_copy(feat_v.at[k], canvas.at[idx_v.at[k]], sem_s).wait()

    load(0, 0)
    load(1, 1)
    for j in range(NCHUNK):
      k = j % 2
      wait_load(j, k)
      compute_idx(k)
      scatter_and_wait(k)
      if j + 2 < NCHUNK:
        load(j + 2, k)


def _sc_scatter(f0, i0, f1, i1, canvas_zeros):
  mesh = plsc.VectorSubcoreMesh(core_axis_name="c", subcore_axis_name="s")
  run = _mpmd._mpmd_map(
      [(mesh, _sc_scatter_body)],
      jax.ShapeDtypeStruct((CANVAS_ROWS, C), jnp.float32),
      input_output_aliases={4: 0},
      compiler_params=pltpu.CompilerParams(
          needs_layout_passes=False, use_tc_tiling_on_sc=False),
      scratch_types=[
          pltpu.VMEM((2, CH * 4), jnp.int32),
          pltpu.VMEM((2, CH), jnp.int32),
          pltpu.VMEM((2, CH, C), jnp.float32),
          pltpu.SemaphoreType.DMA,
          pltpu.SemaphoreType.DMA,
      ],
  )
  return run(f0, i0, f1, i1, canvas_zeros)


S_BLK = 4096
NSB = NY * NX // S_BLK
Y_BLK = S_BLK // NX


def _tc_transpose_body(c0_ref, c1_ref, o0_ref, o1_ref):
  for c_ref, o_ref in ((c0_ref, o0_ref), (c1_ref, o1_ref)):
    x = c_ref[...]
    for b in range(B):
      half = x[:, b * C:(b + 1) * C]
      for yy in range(Y_BLK):
        o_ref[b, :, yy, :] = jnp.transpose(
            half[yy * NX:(yy + 1) * NX, :], (1, 0))


def _tc_transpose(canvas2d):
  in_spec0 = pl.BlockSpec((S_BLK, 2 * C), lambda s: (s, 0))
  in_spec1 = pl.BlockSpec((S_BLK, 2 * C), lambda s: (s + NSB, 0))
  out_spec = pl.BlockSpec((B, C, Y_BLK, NX), lambda s: (0, 0, s, 0))
  return pl.pallas_call(
      _tc_transpose_body,
      grid=(NSB,),
      in_specs=[in_spec0, in_spec1],
      out_specs=[out_spec, out_spec],
      out_shape=[
          jax.ShapeDtypeStruct((B, C, NY, NX), jnp.float32),
          jax.ShapeDtypeStruct((B, C, NY, NX), jnp.float32),
      ],
      compiler_params=pltpu.CompilerParams(
          dimension_semantics=("parallel",),
      ),
  )(canvas2d, canvas2d)


def kernel(pillar_features_bin_0, voxel_coords_bin_0, pillar_features_bin_1,
           voxel_coords_bin_1):
  zeros = jnp.zeros((NBIN * NY * NX, 2 * C), jnp.float32)
  canvas = _sc_scatter(pillar_features_bin_0, voxel_coords_bin_0.reshape(-1),
                       pillar_features_bin_1, voxel_coords_bin_1.reshape(-1),
                       zeros.reshape(CANVAS_ROWS, C))
  return _tc_transpose(canvas.reshape(NBIN * NY * NX, 2 * C))
